# Initial kernel scaffold; baseline (speedup 1.0000x reference)
import functools

import jax
import jax.numpy as jnp
from jax import lax
from jax.experimental import pallas as pl
from jax.experimental.pallas import tpu as pltpu
from jax.experimental.pallas import tpu_sc as plsc

N = 4096
MAX_NN = 32
CUTOFF2 = jnp.float32(100.0)
BIG = jnp.float32(1e10)
BIGH = jnp.float32(5e9)
NW = 32
RPW = N // NW
PAD = 16
CAND = N + 64

_mesh = plsc.VectorSubcoreMesh(core_axis_name="c", subcore_axis_name="s")


@functools.partial(
    pl.kernel,
    mesh=_mesh,
    out_type=[
        jax.ShapeDtypeStruct((N * MAX_NN,), jnp.int32),
        jax.ShapeDtypeStruct((N * MAX_NN,), jnp.float32),
    ],
    scratch_types=[
        pltpu.VMEM((N + 16,), jnp.float32),
        pltpu.VMEM((N + 16,), jnp.float32),
        pltpu.VMEM((N + 16,), jnp.float32),
        pltpu.VMEM((PAD + N + 16,), jnp.int32),
        pltpu.VMEM((CAND,), jnp.float32),
        pltpu.VMEM((CAND,), jnp.int32),
        pltpu.VMEM((RPW * MAX_NN,), jnp.int32),
        pltpu.VMEM((RPW * MAX_NN,), jnp.float32),
    ],
)
def _sc_topk(px_h, py_h, pz_h, b_h, src_o, d2_o,
             px_v, py_v, pz_v, b_v, cd_v, ci_v, os_v, od_v):
    wid = lax.axis_index("s") * 2 + lax.axis_index("c")
    base = wid * RPW

    pltpu.sync_copy(px_h, px_v.at[pl.ds(0, N)])
    pltpu.sync_copy(py_h, py_v.at[pl.ds(0, N)])
    pltpu.sync_copy(pz_h, pz_v.at[pl.ds(0, N)])
    pltpu.sync_copy(b_h, b_v.at[pl.ds(PAD, N)])
    b_v[pl.ds(0, 16)] = jnp.full((16,), -1, jnp.int32)
    b_v[pl.ds(PAD + N, 16)] = jnp.full((16,), 2**30, jnp.int32)

    iota = lax.broadcasted_iota(jnp.int32, (16,), 0)
    bigv = jnp.full((16,), BIG, jnp.float32)

    b0 = b_v[PAD + base]

    def back_cond(c):
        return c[1] == b0

    def back_body(c):
        s = c[0] - 1
        return (s, b_v[PAD + s - 1])

    s0, _ = lax.while_loop(back_cond, back_body, (base, b_v[PAD + base - 1]))

    def row_body(il, carry):
        s, e = carry
        ia = base + il
        bi = b_v[PAD + ia]
        xi = px_v[ia]
        yi = py_v[ia]
        zi = pz_v[ia]

        def sc_(c):
            return c[1] < bi

        def sb_(c):
            s_ = c[0] + 1
            return (s_, b_v[PAD + s_])

        s, _ = lax.while_loop(sc_, sb_, (s, b_v[PAD + s]))

        def ec_(c):
            return c[1] <= bi

        def eb_(c):
            e_ = c[0] + 1
            return (e_, b_v[PAD + e_])

        e, _ = lax.while_loop(ec_, eb_, (e, b_v[PAD + e]))

        c0 = s // 16
        c1 = (e + 15) // 16

        def chunk_body(c, cnt):
            j0 = c * 16
            jv = j0 + iota
            bj = b_v[pl.ds(PAD + j0, 16)]
            xj = px_v[pl.ds(j0, 16)]
            yj = py_v[pl.ds(j0, 16)]
            zj = pz_v[pl.ds(j0, 16)]
            dx = xi - xj
            dy = yi - yj
            dz = zi - zj
            d2 = dx * dx + dy * dy + dz * dz
            valid = (bj == bi) & (jv != ia) & (d2 <= CUTOFF2)
            plsc.store_compressed(cd_v.at[pl.ds(cnt, 16)], d2, valid)
            plsc.store_compressed(ci_v.at[pl.ds(cnt, 16)], jv, valid)
            return cnt + plsc.all_reduce_population_count(valid)[0]

        cnt = lax.fori_loop(c0, c1, chunk_body, jnp.int32(0))

        cd_v[pl.ds(cnt, 16)] = bigv
        cd_v[pl.ds(cnt + 16, 16)] = bigv

        ad = cd_v[pl.ds(0, 16)]
        ai = ci_v[pl.ds(0, 16)]
        bd = cd_v[pl.ds(16, 16)]
        bv = ci_v[pl.ds(16, 16)]
        ad, ai = plsc.sort_key_val(ad, ai)
        bd, bv = plsc.sort_key_val(bd, bv)
        rbd = lax.rev(bd, (0,))
        rbv = lax.rev(bv, (0,))
        m = ad <= rbd
        ld = jnp.where(m, ad, rbd)
        li = jnp.where(m, ai, rbv)
        hd = jnp.where(m, rbd, ad)
        hi = jnp.where(m, rbv, ai)
        ad, ai = plsc.sort_key_val(ld, li)
        bd, bv = plsc.sort_key_val(hd, hi)

        def tail_body(k, c):
            ad, ai, bd, bv = c
            lane = k * 16 + iota
            cd = cd_v[pl.ds(k * 16, 16)]
            ci = ci_v[pl.ds(k * 16, 16)]
            cd = jnp.where(lane < cnt, cd, BIG)
            cd, ci = plsc.sort_key_val(cd, ci)
            rcd = lax.rev(cd, (0,))
            rci = lax.rev(ci, (0,))
            m = ad <= rcd
            ld = jnp.where(m, ad, rcd)
            li = jnp.where(m, ai, rci)
            hd = jnp.where(m, rcd, ad)
            hi = jnp.where(m, rci, ai)
            ad2, ai2 = plsc.sort_key_val(ld, li)
            hd, hi = plsc.sort_key_val(hd, hi)
            rhd = lax.rev(hd, (0,))
            rhi = lax.rev(hi, (0,))
            m2 = bd <= rhd
            l2d = jnp.where(m2, bd, rhd)
            l2i = jnp.where(m2, bv, rhi)
            bd2, bv2 = plsc.sort_key_val(l2d, l2i)
            return (ad2, ai2, bd2, bv2)

        nch = (cnt + 15) // 16
        ad, ai, bd, bv = lax.fori_loop(2, nch, tail_body, (ad, ai, bd, bv))

        sa = jnp.where(ad < BIGH, ai, ia)
        sb = jnp.where(bd < BIGH, bv, ia)
        os_v[pl.ds(il * MAX_NN, 16)] = sa
        os_v[pl.ds(il * MAX_NN + 16, 16)] = sb
        od_v[pl.ds(il * MAX_NN, 16)] = ad
        od_v[pl.ds(il * MAX_NN + 16, 16)] = bd
        return (s, e)

    lax.fori_loop(0, RPW, row_body, (s0, s0))

    pltpu.sync_copy(os_v, src_o.at[pl.ds(base * MAX_NN, RPW * MAX_NN)])
    pltpu.sync_copy(od_v, d2_o.at[pl.ds(base * MAX_NN, RPW * MAX_NN)])


def kernel(pos, batch):
    n = pos.shape[0]
    px = pos[:, 0]
    py = pos[:, 1]
    pz = pos[:, 2]
    b32 = batch.astype(jnp.int32)
    src_flat, d2_flat = _sc_topk(px, py, pz, b32)
    col = jnp.broadcast_to(
        jnp.arange(n, dtype=jnp.int32)[:, None], (n, MAX_NN)
    ).reshape(-1)
    edge_index = jnp.stack([src_flat, col], axis=0)
    ev = d2_flat < BIGH
    edge_weight = jnp.sqrt(jnp.where(ev, d2_flat, jnp.float32(1.0)))
    edge_weight = edge_weight * ev.astype(jnp.float32)
    return edge_index, edge_weight


# speedup vs baseline: 131.3586x; 131.3586x over previous
import functools

import jax
import jax.numpy as jnp
from jax import lax
from jax.experimental import pallas as pl
from jax.experimental.pallas import tpu as pltpu
from jax.experimental.pallas import tpu_sc as plsc

N = 4096
MAX_NN = 32
CUTOFF2 = 100.0
BIG = 1e10
BIGH = 5e9
NW = 32
RPW = N // NW
PAD = 16
CAND = N + 64
NUM_MOL = 128

_mesh = plsc.VectorSubcoreMesh(core_axis_name="c", subcore_axis_name="s")


def _sget(ref, idx):
    return ref[pl.ds(idx, 16)][0]

@functools.partial(
    pl.kernel,
    mesh=_mesh,
    compiler_params=pltpu.CompilerParams(needs_layout_passes=False),
    out_type=[
        jax.ShapeDtypeStruct((N * MAX_NN,), jnp.int32),
        jax.ShapeDtypeStruct((N * MAX_NN,), jnp.float32),
    ],
    scratch_types=[
        pltpu.VMEM((N + 16,), jnp.float32),
        pltpu.VMEM((N + 16,), jnp.float32),
        pltpu.VMEM((N + 16,), jnp.float32),
        pltpu.VMEM((PAD + N + 16,), jnp.int32),
        pltpu.VMEM((CAND,), jnp.float32),
        pltpu.VMEM((CAND,), jnp.int32),
        pltpu.VMEM((RPW * MAX_NN,), jnp.int32),
        pltpu.VMEM((RPW * MAX_NN,), jnp.float32),
        pltpu.VMEM((NUM_MOL + 16,), jnp.int32),
        pltpu.VMEM((NUM_MOL + 16,), jnp.int32),
    ],
)
def _sc_topk(px_h, py_h, pz_h, b_h, src_o, d2_o,
             px_v, py_v, pz_v, b_v, cd_v, ci_v, os_v, od_v, ss_v, se_v):
    wid = lax.axis_index("s") * 2 + lax.axis_index("c")
    base = wid * RPW

    pltpu.sync_copy(px_h, px_v.at[pl.ds(0, N)])
    pltpu.sync_copy(py_h, py_v.at[pl.ds(0, N)])
    pltpu.sync_copy(pz_h, pz_v.at[pl.ds(0, N)])
    pltpu.sync_copy(b_h, b_v.at[pl.ds(PAD, N)])
    b_v[pl.ds(0, 16)] = jnp.full((16,), -1, jnp.int32)
    b_v[pl.ds(PAD + N, 16)] = jnp.full((16,), 2**30, jnp.int32)

    iota = lax.broadcasted_iota(jnp.int32, (16,), 0)
    bigv = jnp.full((16,), BIG, jnp.float32)

    def seg_body(c, acc):
        j0 = c * 16
        jv = j0 + iota
        bj = b_v[pl.ds(PAD + j0, 16)]
        bp = b_v[pl.ds(PAD + j0 - 1, 16)]
        bn = b_v[pl.ds(PAD + j0 + 1, 16)]
        plsc.store_scatter(ss_v, [bj], jv, mask=bj != bp)
        plsc.store_scatter(se_v, [bj], jv + 1, mask=bj != bn)
        return acc

    lax.fori_loop(0, N // 16, seg_body, jnp.int32(0))

    def row_body(il, acc):
        ia = base + il
        bi = _sget(b_v, PAD + ia)
        xi = _sget(px_v, ia)
        yi = _sget(py_v, ia)
        zi = _sget(pz_v, ia)
        s = _sget(ss_v, bi)
        e = _sget(se_v, bi)

        c0 = s // 16
        c1 = (e + 15) // 16

        def chunk_body(c, cnt):
            j0 = c * 16
            jv = j0 + iota
            bj = b_v[pl.ds(PAD + j0, 16)]
            xj = px_v[pl.ds(j0, 16)]
            yj = py_v[pl.ds(j0, 16)]
            zj = pz_v[pl.ds(j0, 16)]
            dx = xi - xj
            dy = yi - yj
            dz = zi - zj
            d2 = dx * dx + dy * dy + dz * dz
            valid = (bj == bi) & (jv != ia) & (d2 <= CUTOFF2)
            plsc.store_compressed(cd_v.at[pl.ds(cnt, 16)], d2, mask=valid)
            plsc.store_compressed(ci_v.at[pl.ds(cnt, 16)], jv, mask=valid)
            return cnt + plsc.all_reduce_population_count(valid)[0]

        cnt = lax.fori_loop(c0, c1, chunk_body, jnp.int32(0))

        cd_v[pl.ds(cnt, 16)] = bigv
        cd_v[pl.ds(cnt + 16, 16)] = bigv

        ad = cd_v[pl.ds(0, 16)]
        ai = ci_v[pl.ds(0, 16)]
        bd = cd_v[pl.ds(16, 16)]
        bv = ci_v[pl.ds(16, 16)]
        ad, ai = plsc.sort_key_val(ad, ai)
        bd, bv = plsc.sort_key_val(bd, bv)
        rbd = lax.rev(bd, (0,))
        rbv = lax.rev(bv, (0,))
        m = ad <= rbd
        ld = jnp.where(m, ad, rbd)
        li = jnp.where(m, ai, rbv)
        hd = jnp.where(m, rbd, ad)
        hi = jnp.where(m, rbv, ai)
        ad, ai = plsc.sort_key_val(ld, li)
        bd, bv = plsc.sort_key_val(hd, hi)

        def tail_body(k, c):
            ad, ai, bd, bv = c
            lane = k * 16 + iota
            cd = cd_v[pl.ds(k * 16, 16)]
            ci = ci_v[pl.ds(k * 16, 16)]
            cd = jnp.where(lane < cnt, cd, BIG)
            cd, ci = plsc.sort_key_val(cd, ci)
            rcd = lax.rev(cd, (0,))
            rci = lax.rev(ci, (0,))
            m = ad <= rcd
            ld = jnp.where(m, ad, rcd)
            li = jnp.where(m, ai, rci)
            hd = jnp.where(m, rcd, ad)
            hi = jnp.where(m, rci, ai)
            ad2, ai2 = plsc.sort_key_val(ld, li)
            hd, hi = plsc.sort_key_val(hd, hi)
            rhd = lax.rev(hd, (0,))
            rhi = lax.rev(hi, (0,))
            m2 = bd <= rhd
            l2d = jnp.where(m2, bd, rhd)
            l2i = jnp.where(m2, bv, rhi)
            bd2, bv2 = plsc.sort_key_val(l2d, l2i)
            return (ad2, ai2, bd2, bv2)

        nch = (cnt + 15) // 16
        ad, ai, bd, bv = lax.fori_loop(2, nch, tail_body, (ad, ai, bd, bv))

        sa = jnp.where(ad < BIGH, ai, ia)
        sb = jnp.where(bd < BIGH, bv, ia)
        os_v[pl.ds(il * MAX_NN, 16)] = sa
        os_v[pl.ds(il * MAX_NN + 16, 16)] = sb
        od_v[pl.ds(il * MAX_NN, 16)] = ad
        od_v[pl.ds(il * MAX_NN + 16, 16)] = bd
        return acc

    lax.fori_loop(0, RPW, row_body, jnp.int32(0))

    pltpu.sync_copy(os_v, src_o.at[pl.ds(base * MAX_NN, RPW * MAX_NN)])
    pltpu.sync_copy(od_v, d2_o.at[pl.ds(base * MAX_NN, RPW * MAX_NN)])


def kernel(pos, batch):
    n = pos.shape[0]
    px = pos[:, 0]
    py = pos[:, 1]
    pz = pos[:, 2]
    b32 = batch.astype(jnp.int32)
    src_flat, d2_flat = _sc_topk(px, py, pz, b32)
    col = jnp.broadcast_to(
        jnp.arange(n, dtype=jnp.int32)[:, None], (n, MAX_NN)
    ).reshape(-1)
    edge_index = jnp.stack([src_flat, col], axis=0)
    ev = d2_flat < BIGH
    edge_weight = jnp.sqrt(jnp.where(ev, d2_flat, 1.0))
    edge_weight = edge_weight * ev.astype(jnp.float32)
    return edge_index, edge_weight


# speedup vs baseline: 131.5503x; 1.0015x over previous
import functools

import jax
import jax.numpy as jnp
from jax import lax
from jax.experimental import pallas as pl
from jax.experimental.pallas import tpu as pltpu
from jax.experimental.pallas import tpu_sc as plsc

N = 4096
MAX_NN = 32
NE = N * MAX_NN
CUTOFF2 = 100.0
BIG = 1e10
BIGH = 5e9
NW = 32
RPW = N // NW
PAD = 16
CAND = N + 64
NUM_MOL = 128

_mesh = plsc.VectorSubcoreMesh(core_axis_name="c", subcore_axis_name="s")


def _sget(ref, idx):
    return ref[pl.ds(idx, 16)][0]


@functools.partial(
    pl.kernel,
    mesh=_mesh,
    compiler_params=pltpu.CompilerParams(needs_layout_passes=False),
    out_type=[
        jax.ShapeDtypeStruct((2 * NE,), jnp.int32),
        jax.ShapeDtypeStruct((NE,), jnp.float32),
    ],
    scratch_types=[
        pltpu.VMEM((N + 16,), jnp.float32),
        pltpu.VMEM((N + 16,), jnp.float32),
        pltpu.VMEM((N + 16,), jnp.float32),
        pltpu.VMEM((PAD + N + 16,), jnp.int32),
        pltpu.VMEM((CAND,), jnp.float32),
        pltpu.VMEM((CAND,), jnp.int32),
        pltpu.VMEM((RPW * MAX_NN,), jnp.int32),
        pltpu.VMEM((RPW * MAX_NN,), jnp.int32),
        pltpu.VMEM((RPW * MAX_NN,), jnp.float32),
        pltpu.VMEM((NUM_MOL + 16,), jnp.int32),
        pltpu.VMEM((NUM_MOL + 16,), jnp.int32),
    ],
)
def _sc_topk(px_h, py_h, pz_h, b_h, ei_o, w_o,
             px_v, py_v, pz_v, b_v, cd_v, ci_v, os_v, oc_v, ow_v, ss_v, se_v):
    wid = lax.axis_index("s") * 2 + lax.axis_index("c")
    base = wid * RPW

    pltpu.sync_copy(px_h, px_v.at[pl.ds(0, N)])
    pltpu.sync_copy(py_h, py_v.at[pl.ds(0, N)])
    pltpu.sync_copy(pz_h, pz_v.at[pl.ds(0, N)])
    pltpu.sync_copy(b_h, b_v.at[pl.ds(PAD, N)])
    b_v[pl.ds(0, 16)] = jnp.full((16,), -1, jnp.int32)
    b_v[pl.ds(PAD + N, 16)] = jnp.full((16,), 2**30, jnp.int32)

    iota = lax.broadcasted_iota(jnp.int32, (16,), 0)
    bigv = jnp.full((16,), BIG, jnp.float32)

    def seg_body(c, acc):
        j0 = c * 16
        jv = j0 + iota
        bj = b_v[pl.ds(PAD + j0, 16)]
        bp = b_v[pl.ds(PAD + j0 - 1, 16)]
        mk = bj != bp
        plsc.store_scatter(ss_v, [bj], jv, mask=mk & (jv < N))
        plsc.store_scatter(se_v, [bp], jv, mask=mk & (jv > 0))
        return acc

    lax.fori_loop(0, N // 16 + 1, seg_body, jnp.int32(0))

    def row_body(il, acc):
        ia = base + il
        bi = _sget(b_v, PAD + ia)
        xi = _sget(px_v, ia)
        yi = _sget(py_v, ia)
        zi = _sget(pz_v, ia)
        s = _sget(ss_v, bi)
        e = _sget(se_v, bi)

        def chunk_body(t, cnt):
            j0 = s + t * 16
            jv = j0 + iota
            xj = px_v[pl.ds(j0, 16)]
            yj = py_v[pl.ds(j0, 16)]
            zj = pz_v[pl.ds(j0, 16)]
            dx = xi - xj
            dy = yi - yj
            dz = zi - zj
            d2 = dx * dx + dy * dy + dz * dz
            valid = (jv < e) & (jv != ia) & (d2 <= CUTOFF2)
            plsc.store_compressed(cd_v.at[pl.ds(cnt, 16)], d2, mask=valid)
            plsc.store_compressed(ci_v.at[pl.ds(cnt, 16)], jv, mask=valid)
            return cnt + plsc.all_reduce_population_count(valid)[0]

        nc = (e - s + 15) // 16
        cnt = lax.fori_loop(0, nc, chunk_body, jnp.int32(0))

        cd_v[pl.ds(cnt, 16)] = bigv
        cd_v[pl.ds(cnt + 16, 16)] = bigv

        ad = cd_v[pl.ds(0, 16)]
        ai = ci_v[pl.ds(0, 16)]
        bd = cd_v[pl.ds(16, 16)]
        bv = ci_v[pl.ds(16, 16)]
        ad, ai = plsc.sort_key_val(ad, ai)
        bd, bv = plsc.sort_key_val(bd, bv)
        rbd = lax.rev(bd, (0,))
        rbv = lax.rev(bv, (0,))
        m = ad <= rbd
        ld = jnp.where(m, ad, rbd)
        li = jnp.where(m, ai, rbv)
        hd = jnp.where(m, rbd, ad)
        hi = jnp.where(m, rbv, ai)
        ad, ai = plsc.sort_key_val(ld, li)
        bd, bv = plsc.sort_key_val(hd, hi)

        def tail_body(k, c):
            ad, ai, bd, bv = c
            lane = k * 16 + iota
            cd = cd_v[pl.ds(k * 16, 16)]
            ci = ci_v[pl.ds(k * 16, 16)]
            cd = jnp.where(lane < cnt, cd, BIG)
            cd, ci = plsc.sort_key_val(cd, ci)
            rcd = lax.rev(cd, (0,))
            rci = lax.rev(ci, (0,))
            m = ad <= rcd
            ld = jnp.where(m, ad, rcd)
            li = jnp.where(m, ai, rci)
            hd = jnp.where(m, rcd, ad)
            hi = jnp.where(m, rci, ai)
            ad2, ai2 = plsc.sort_key_val(ld, li)
            hd, hi = plsc.sort_key_val(hd, hi)
            rhd = lax.rev(hd, (0,))
            rhi = lax.rev(hi, (0,))
            m2 = bd <= rhd
            l2d = jnp.where(m2, bd, rhd)
            l2i = jnp.where(m2, bv, rhi)
            bd2, bv2 = plsc.sort_key_val(l2d, l2i)
            return (ad2, ai2, bd2, bv2)

        nch = (cnt + 15) // 16
        ad, ai, bd, bv = lax.fori_loop(2, nch, tail_body, (ad, ai, bd, bv))

        def weight(d2k, valid):
            ib = plsc.bitcast(d2k, jnp.int32)
            y = plsc.bitcast(
                jnp.int32(0x5F3759DF) - lax.shift_right_logical(ib, 1),
                jnp.float32,
            )
            h = 0.5 * d2k
            y = y * (1.5 - h * y * y)
            y = y * (1.5 - h * y * y)
            y = y * (1.5 - h * y * y)
            return jnp.where(valid, d2k * y, 0.0)

        va = ad < BIGH
        vb = bd < BIGH
        sa = jnp.where(va, ai, ia)
        sb = jnp.where(vb, bv, ia)
        dstv = jnp.broadcast_to(ia, (16,))
        os_v[pl.ds(il * MAX_NN, 16)] = sa
        os_v[pl.ds(il * MAX_NN + 16, 16)] = sb
        oc_v[pl.ds(il * MAX_NN, 16)] = dstv
        oc_v[pl.ds(il * MAX_NN + 16, 16)] = dstv
        ow_v[pl.ds(il * MAX_NN, 16)] = weight(ad, va)
        ow_v[pl.ds(il * MAX_NN + 16, 16)] = weight(bd, vb)
        return acc

    lax.fori_loop(0, RPW, row_body, jnp.int32(0))

    pltpu.sync_copy(os_v, ei_o.at[pl.ds(base * MAX_NN, RPW * MAX_NN)])
    pltpu.sync_copy(oc_v, ei_o.at[pl.ds(NE + base * MAX_NN, RPW * MAX_NN)])
    pltpu.sync_copy(ow_v, w_o.at[pl.ds(base * MAX_NN, RPW * MAX_NN)])


def kernel(pos, batch):
    px = pos[:, 0]
    py = pos[:, 1]
    pz = pos[:, 2]
    b32 = batch.astype(jnp.int32)
    ei_flat, edge_weight = _sc_topk(px, py, pz, b32)
    return ei_flat.reshape(2, NE), edge_weight
